# baseline (speedup 1.0000x reference)
import dataclasses
import functools

import jax
import jax.numpy as jnp
from jax import lax
from jax.experimental import pallas as pl
from jax.experimental.pallas import tpu as pltpu
from jax.experimental.pallas import tpu_sc as plsc

B, S, D = 32, 2048, 256
TOPK = 3
BB = 2
NSUB = 16


def _routing_kernel(gw_r_ref, gw_i_ref, wg_ref, bg_ref, wa_t_ref, ba_ref,
                    ww_ref, gate_ref, ent_ref):
    flat = jnp.concatenate([gw_r_ref[...], gw_i_ref[...]], axis=1)
    gate_logit = jnp.sum(flat * wg_ref[...], axis=1, keepdims=True) + bg_ref[0, 0]
    gate_ref[...] = jax.nn.sigmoid(gate_logit)
    logits = jnp.dot(flat, wa_t_ref[...],
                     preferred_element_type=jnp.float32) + ba_ref[...]
    m = jnp.max(logits, axis=1, keepdims=True)
    e = jnp.exp(logits - m)
    ww = e / jnp.sum(e, axis=1, keepdims=True)
    ww_ref[...] = ww
    ent = jnp.sum(-(ww * jnp.log(ww + 1e-10)), axis=1, keepdims=True)
    ent_ref[...] = jnp.sum(ent, axis=0, keepdims=True) * (1.0 / B)


def _sc_topk_body(ww_hbm, gate_hbm, iota_hbm, idx_hbm, euv_hbm,
                  row, iot, gate_v, bv, bi, oi, ov, sem):
    c = lax.axis_index("c")
    s = lax.axis_index("s")
    b = c * NSUB + s
    pltpu.async_copy(ww_hbm.at[pl.ds(b, 1), :], row, sem).wait()
    pltpu.async_copy(iota_hbm, iot, sem).wait()
    pltpu.async_copy(gate_hbm, gate_v, sem).wait()

    NEG = -1.0
    BIGI = 1.0e9
    found_v, found_i = [], []
    for k in range(TOPK):
        bv[...] = jnp.full((NSUB,), NEG, jnp.float32)
        bi[...] = jnp.full((NSUB,), BIGI, jnp.float32)
        prev = list(found_i)

        @pl.loop(0, S // NSUB)
        def _(i):
            chunk = row[0, pl.ds(i * NSUB, NSUB)]
            ich = iot[0, pl.ds(i * NSUB, NSUB)]
            for fprev in prev:
                chunk = jnp.where(ich == fprev, NEG, chunk)
            upd = chunk > bv[...]
            bv[...] = jnp.where(upd, chunk, bv[...])
            bi[...] = jnp.where(upd, ich, bi[...])

        v16 = bv[...]
        i16 = bi[...]
        m = jnp.max(v16)
        fi = jnp.min(jnp.where(v16 == m, i16, BIGI))
        found_v.append(m)
        found_i.append(fi)

    lane = iot[0, 0:NSUB]
    gchunk = gate_v[0, pl.ds((b // NSUB) * NSUB, NSUB)]
    bmod = lax.convert_element_type(b % NSUB, jnp.float32)
    g = jnp.sum(jnp.where(lane == bmod, gchunk, 0.0))
    denom = found_v[0] + found_v[1] + found_v[2] + 1e-6
    scale_v = jnp.full((NSUB,), g) / jnp.full((NSUB,), denom)
    oi_v = jnp.zeros((NSUB,), jnp.float32)
    val_v = jnp.zeros((NSUB,), jnp.float32)
    for k in range(TOPK):
        oi_v = jnp.where(lane == float(k), found_i[k], oi_v)
        val_v = jnp.where(lane == float(k), found_v[k], val_v)
    oi[0, :] = oi_v
    ov[0, :] = val_v * scale_v
    pltpu.async_copy(oi, idx_hbm.at[pl.ds(b, 1), :], sem).wait()
    pltpu.async_copy(ov, euv_hbm.at[pl.ds(b, 1), :], sem).wait()


def _sc_topk(ww, gate, iota_row, interpret=False):
    f32 = jnp.float32
    cp = pltpu.CompilerParams()
    if "needs_layout_passes" in pltpu.CompilerParams.__dataclass_fields__:
        cp = dataclasses.replace(cp, needs_layout_passes=False)
    fn = pl.kernel(
        _sc_topk_body,
        out_type=(jax.ShapeDtypeStruct((B, NSUB), f32),
                  jax.ShapeDtypeStruct((B, NSUB), f32)),
        mesh=plsc.VectorSubcoreMesh(core_axis_name="c", subcore_axis_name="s",
                            num_cores=2, num_subcores=NSUB),
        scratch_types=[
            pltpu.VMEM((1, S), f32),
            pltpu.VMEM((1, S), f32),
            pltpu.VMEM((1, B), f32),
            pltpu.VMEM((NSUB,), f32),
            pltpu.VMEM((NSUB,), f32),
            pltpu.VMEM((1, NSUB), f32),
            pltpu.VMEM((1, NSUB), f32),
            pltpu.SemaphoreType.DMA,
        ],
        compiler_params=cp,
        interpret=interpret,
    )
    return fn(ww, gate, iota_row)


def _stream_kernel(q_r_ref, q_i_ref, g_r_ref, b_r_ref, g_i_ref, b_i_ref,
                   mem_r_ref, mem_i_ref,
                   read_r_ref, read_i_ref, next_r_ref, next_i_ref):
    def _ln_dense(x, gamma, beta):
        mu = jnp.mean(x, axis=1, keepdims=True)
        var = jnp.mean(x * x, axis=1, keepdims=True) - mu * mu
        rg = lax.rsqrt(var + 1e-5)
        h = -(mu * rg)
        return (x * rg + h) * gamma + beta

    for ib in range(BB):
        mem_r = mem_r_ref[ib]
        mem_i = mem_i_ref[ib]
        q_r = q_r_ref[ib]
        q_i = q_i_ref[ib]

        sim = jnp.sum(mem_r * q_r + mem_i * q_i, axis=1, keepdims=True)
        p = jnp.exp(sim - jnp.max(sim))
        inv_l = 1.0 / jnp.sum(p)
        read_r_ref[ib] = jnp.sum(p * mem_r, axis=0, keepdims=True) * inv_l
        read_i_ref[ib] = jnp.sum(p * mem_i, axis=0, keepdims=True) * inv_l

        next_r_ref[ib] = _ln_dense(mem_r, g_r_ref[...], b_r_ref[...])
        next_i_ref[ib] = _ln_dense(mem_i, g_i_ref[...], b_i_ref[...])


def _fix_kernel(idx_ref, euv_ref,
                q_r_ref, q_i_ref, g_r_ref, b_r_ref, g_i_ref, b_i_ref,
                prev_r_hbm, prev_i_hbm, next_r_in, next_i_in,
                next_r_out, next_i_out,
                rows_r, rows_i, out_r, out_i, sem):
    del next_r_in, next_i_in
    copies = []
    for j in range(B * TOPK):
        b, k = j // TOPK, j % TOPK
        i = idx_ref[b, k]
        cr = pltpu.make_async_copy(prev_r_hbm.at[b, pl.ds(i, 1), :],
                                   rows_r.at[pl.ds(j, 1), :], sem)
        ci = pltpu.make_async_copy(prev_i_hbm.at[b, pl.ds(i, 1), :],
                                   rows_i.at[pl.ds(j, 1), :], sem)
        cr.start()
        ci.start()
        copies.append((cr, ci))

    def _ln_row(x, gamma, beta):
        mu = jnp.mean(x, axis=1, keepdims=True)
        xc = x - mu
        var = jnp.mean(xc * xc, axis=1, keepdims=True)
        return xc * lax.rsqrt(var + 1e-5) * gamma + beta

    for j in range(B * TOPK):
        b, k = j // TOPK, j % TOPK
        cr, ci = copies[j]
        cr.wait()
        ci.wait()
        e = euv_ref[b, k]
        q_r = q_r_ref[b]
        q_i = q_i_ref[b]
        row_r = rows_r[pl.ds(j, 1), :]
        row_i = rows_i[pl.ds(j, 1), :]
        nr = row_r + e * (q_r - row_r)
        ni = row_i + e * (q_i - row_i)
        out_r[pl.ds(j, 1), :] = _ln_row(nr, g_r_ref[...], b_r_ref[...])
        out_i[pl.ds(j, 1), :] = _ln_row(ni, g_i_ref[...], b_i_ref[...])

    outc = []
    for j in range(B * TOPK):
        b, k = j // TOPK, j % TOPK
        i = idx_ref[b, k]
        cr = pltpu.make_async_copy(out_r.at[pl.ds(j, 1), :],
                                   next_r_out.at[b, pl.ds(i, 1), :], sem)
        ci = pltpu.make_async_copy(out_i.at[pl.ds(j, 1), :],
                                   next_i_out.at[b, pl.ds(i, 1), :], sem)
        cr.start()
        ci.start()
        outc.append((cr, ci))
    for cr, ci in outc:
        cr.wait()
        ci.wait()


@functools.partial(jax.jit, static_argnames=("interpret",))
def kernel(gw_state_real, gw_state_imag, prev_mem_real, prev_mem_imag,
           Wg, bg, Wa, ba, gamma_r, beta_r, gamma_i, beta_i, interpret=False):
    f32 = jnp.float32
    ww, gate, ent = pl.pallas_call(
        _routing_kernel,
        out_shape=(jax.ShapeDtypeStruct((B, S), f32),
                   jax.ShapeDtypeStruct((B, 1), f32),
                   jax.ShapeDtypeStruct((1, 1), f32)),
        interpret=interpret,
    )(gw_state_real, gw_state_imag, Wg, bg.reshape(1, 1), Wa.T,
      ba.reshape(1, S))

    iota_row = jnp.arange(S, dtype=f32).reshape(1, S)
    idxf, euv16 = _sc_topk(ww, gate.reshape(1, B), iota_row, interpret=interpret)
    idx = idxf[:, :TOPK].astype(jnp.int32)
    euv = euv16[:, :TOPK]

    q_r = gw_state_real.reshape(B, 1, D)
    q_i = gw_state_imag.reshape(B, 1, D)
    g_r, b_r = gamma_r.reshape(1, D), beta_r.reshape(1, D)
    g_i, b_i = gamma_i.reshape(1, D), beta_i.reshape(1, D)

    read_r, read_i, next_r, next_i = pl.pallas_call(
        _stream_kernel,
        grid=(B // BB,),
        in_specs=[
            pl.BlockSpec((BB, 1, D), lambda b: (b, 0, 0)),
            pl.BlockSpec((BB, 1, D), lambda b: (b, 0, 0)),
            pl.BlockSpec((1, D), lambda b: (0, 0)),
            pl.BlockSpec((1, D), lambda b: (0, 0)),
            pl.BlockSpec((1, D), lambda b: (0, 0)),
            pl.BlockSpec((1, D), lambda b: (0, 0)),
            pl.BlockSpec((BB, S, D), lambda b: (b, 0, 0)),
            pl.BlockSpec((BB, S, D), lambda b: (b, 0, 0)),
        ],
        out_specs=[
            pl.BlockSpec((BB, 1, D), lambda b: (b, 0, 0)),
            pl.BlockSpec((BB, 1, D), lambda b: (b, 0, 0)),
            pl.BlockSpec((BB, S, D), lambda b: (b, 0, 0)),
            pl.BlockSpec((BB, S, D), lambda b: (b, 0, 0)),
        ],
        out_shape=(jax.ShapeDtypeStruct((B, 1, D), f32),
                   jax.ShapeDtypeStruct((B, 1, D), f32),
                   jax.ShapeDtypeStruct((B, S, D), f32),
                   jax.ShapeDtypeStruct((B, S, D), f32)),
        interpret=interpret,
    )(q_r, q_i, g_r, b_r, g_i, b_i, prev_mem_real, prev_mem_imag)

    smem = pl.BlockSpec(memory_space=pltpu.MemorySpace.SMEM)
    anys = pl.BlockSpec(memory_space=pl.ANY)
    vmem = pl.BlockSpec(memory_space=pltpu.MemorySpace.VMEM)
    next_r2, next_i2 = pl.pallas_call(
        _fix_kernel,
        in_specs=[smem, smem, vmem, vmem, vmem, vmem, vmem, vmem,
                  anys, anys, anys, anys],
        out_specs=[anys, anys],
        out_shape=(jax.ShapeDtypeStruct((B, S, D), f32),
                   jax.ShapeDtypeStruct((B, S, D), f32)),
        input_output_aliases={10: 0, 11: 1},
        scratch_shapes=[pltpu.VMEM((B * TOPK, D), f32),
                        pltpu.VMEM((B * TOPK, D), f32),
                        pltpu.VMEM((B * TOPK, D), f32),
                        pltpu.VMEM((B * TOPK, D), f32),
                        pltpu.SemaphoreType.DMA],
        interpret=interpret,
    )(idx, euv, q_r, q_i, g_r, b_r, g_i, b_i,
      prev_mem_real, prev_mem_imag, next_r, next_i)

    return (read_r.reshape(B, D), read_i.reshape(B, D), next_r2, next_i2,
            ent.reshape(()))


# speedup vs baseline: 1.1259x; 1.1259x over previous
import dataclasses
import functools

import jax
import jax.numpy as jnp
from jax import lax
from jax.experimental import pallas as pl
from jax.experimental.pallas import tpu as pltpu
from jax.experimental.pallas import tpu_sc as plsc

B, S, D = 32, 2048, 256
TOPK = 3
BB = 2
NSUB = 16


def _routing_kernel(gw_r_ref, gw_i_ref, wg_ref, bg_ref, wa_t_ref, ba_ref,
                    ww_ref, gate_ref, ent_ref):
    flat = jnp.concatenate([gw_r_ref[...], gw_i_ref[...]], axis=1)
    gate_logit = jnp.sum(flat * wg_ref[...], axis=1, keepdims=True) + bg_ref[0, 0]
    gate_ref[...] = jax.nn.sigmoid(gate_logit)
    logits = jnp.dot(flat, wa_t_ref[...],
                     preferred_element_type=jnp.float32) + ba_ref[...]
    m = jnp.max(logits, axis=1, keepdims=True)
    e = jnp.exp(logits - m)
    ww = e / jnp.sum(e, axis=1, keepdims=True)
    ww_ref[...] = ww
    ent = jnp.sum(-(ww * jnp.log(ww + 1e-10)), axis=1, keepdims=True)
    ent_ref[...] = jnp.sum(ent, axis=0, keepdims=True) * (1.0 / B)


def _sc_topk_body(ww_hbm, gate_hbm, idx_hbm, euv_hbm,
                  row, gate_v, bv, bi, oi, ov, sem):
    c = lax.axis_index("c")
    s = lax.axis_index("s")
    b = c * NSUB + s
    pltpu.async_copy(ww_hbm.at[pl.ds(b, 1), :], row, sem).wait()
    pltpu.async_copy(gate_hbm, gate_v, sem).wait()

    NEG = -1.0
    BIGI = jnp.int32(1 << 30)
    lane = lax.iota(jnp.int32, NSUB)
    UNROLL = 4
    found_v, found_i = [], []
    for k in range(TOPK):
        bv[...] = jnp.full((NSUB,), NEG, jnp.float32)
        bi[...] = jnp.full((NSUB,), BIGI, jnp.int32)
        prev = list(found_i)

        @pl.loop(0, S // NSUB, step=UNROLL)
        def _(i):
            bvv = bv[...]
            biv = bi[...]
            for u in range(UNROLL):
                base = (i + u) * NSUB
                chunk = row[0, pl.ds(base, NSUB)]
                ich = lane + base
                for fprev in prev:
                    chunk = jnp.where(ich == fprev, NEG, chunk)
                upd = chunk > bvv
                bvv = jnp.where(upd, chunk, bvv)
                biv = jnp.where(upd, ich, biv)
            bv[...] = bvv
            bi[...] = biv

        v16 = bv[...]
        i16 = bi[...]
        m = jnp.max(v16)
        fi = jnp.min(jnp.where(v16 == m, i16, BIGI))
        found_v.append(m)
        found_i.append(fi)

    gchunk = gate_v[0, pl.ds((b // NSUB) * NSUB, NSUB)]
    g = jnp.sum(jnp.where(lane == b % NSUB, gchunk, 0.0))
    denom = found_v[0] + found_v[1] + found_v[2] + 1e-6
    scale_v = jnp.full((NSUB,), g) / jnp.full((NSUB,), denom)
    oi_v = jnp.zeros((NSUB,), jnp.int32)
    val_v = jnp.zeros((NSUB,), jnp.float32)
    for k in range(TOPK):
        oi_v = jnp.where(lane == k, found_i[k], oi_v)
        val_v = jnp.where(lane == k, found_v[k], val_v)
    oi[0, :] = oi_v
    ov[0, :] = val_v * scale_v
    pltpu.async_copy(oi, idx_hbm.at[pl.ds(b, 1), :], sem).wait()
    pltpu.async_copy(ov, euv_hbm.at[pl.ds(b, 1), :], sem).wait()


def _sc_topk(ww, gate, interpret=False):
    f32 = jnp.float32
    cp = pltpu.CompilerParams()
    if "needs_layout_passes" in pltpu.CompilerParams.__dataclass_fields__:
        cp = dataclasses.replace(cp, needs_layout_passes=False)
    fn = pl.kernel(
        _sc_topk_body,
        out_type=(jax.ShapeDtypeStruct((B, NSUB), jnp.int32),
                  jax.ShapeDtypeStruct((B, NSUB), f32)),
        mesh=plsc.VectorSubcoreMesh(core_axis_name="c", subcore_axis_name="s",
                                    num_cores=2, num_subcores=NSUB),
        scratch_types=[
            pltpu.VMEM((1, S), f32),
            pltpu.VMEM((1, B), f32),
            pltpu.VMEM((NSUB,), f32),
            pltpu.VMEM((NSUB,), jnp.int32),
            pltpu.VMEM((1, NSUB), jnp.int32),
            pltpu.VMEM((1, NSUB), f32),
            pltpu.SemaphoreType.DMA,
        ],
        compiler_params=cp,
        interpret=interpret,
    )
    return fn(ww, gate)


def _stream_kernel(idx_ref, euv_ref,
                   q_r_ref, q_i_ref, g_r_ref, b_r_ref, g_i_ref, b_i_ref,
                   mem_r_ref, mem_i_ref,
                   read_r_ref, read_i_ref, next_r_ref, next_i_ref):
    step = pl.program_id(0)

    def _ln_row(x, gamma, beta):
        mu = jnp.mean(x, axis=1, keepdims=True)
        xc = x - mu
        var = jnp.mean(xc * xc, axis=1, keepdims=True)
        return xc * lax.rsqrt(var + 1e-5) * gamma + beta

    def _ln_dense(x, gamma, beta):
        mu = jnp.mean(x, axis=1, keepdims=True)
        var = jnp.mean(x * x, axis=1, keepdims=True) - mu * mu
        rg = lax.rsqrt(var + 1e-5)
        h = -(mu * rg)
        return (x * rg + h) * gamma + beta

    for ib in range(BB):
        mem_r = mem_r_ref[ib]
        mem_i = mem_i_ref[ib]
        q_r = q_r_ref[ib]
        q_i = q_i_ref[ib]

        sim = jnp.sum(mem_r * q_r + mem_i * q_i, axis=1, keepdims=True)
        p = jnp.exp(sim - jnp.max(sim))
        inv_l = 1.0 / jnp.sum(p)
        read_r_ref[ib] = jnp.sum(p * mem_r, axis=0, keepdims=True) * inv_l
        read_i_ref[ib] = jnp.sum(p * mem_i, axis=0, keepdims=True) * inv_l

        next_r_ref[ib] = _ln_dense(mem_r, g_r_ref[...], b_r_ref[...])
        next_i_ref[ib] = _ln_dense(mem_i, g_i_ref[...], b_i_ref[...])

        for k in range(TOPK):
            i = idx_ref[step * BB + ib, k]
            e = euv_ref[step * BB + ib, k]
            row_r = mem_r_ref[ib, pl.ds(i, 1), :]
            row_i = mem_i_ref[ib, pl.ds(i, 1), :]
            nr = row_r + e * (q_r - row_r)
            ni = row_i + e * (q_i - row_i)
            next_r_ref[ib, pl.ds(i, 1), :] = _ln_row(nr, g_r_ref[...], b_r_ref[...])
            next_i_ref[ib, pl.ds(i, 1), :] = _ln_row(ni, g_i_ref[...], b_i_ref[...])


@functools.partial(jax.jit, static_argnames=("interpret",))
def kernel(gw_state_real, gw_state_imag, prev_mem_real, prev_mem_imag,
           Wg, bg, Wa, ba, gamma_r, beta_r, gamma_i, beta_i, interpret=False):
    f32 = jnp.float32
    ww, gate, ent = pl.pallas_call(
        _routing_kernel,
        out_shape=(jax.ShapeDtypeStruct((B, S), f32),
                   jax.ShapeDtypeStruct((B, 1), f32),
                   jax.ShapeDtypeStruct((1, 1), f32)),
        interpret=interpret,
    )(gw_state_real, gw_state_imag, Wg, bg.reshape(1, 1), Wa.T,
      ba.reshape(1, S))

    idx32, euv16 = _sc_topk(ww, gate.reshape(1, B), interpret=interpret)
    idx = idx32[:, :TOPK]
    euv = euv16[:, :TOPK]

    q_r = gw_state_real.reshape(B, 1, D)
    q_i = gw_state_imag.reshape(B, 1, D)

    grid_spec = pltpu.PrefetchScalarGridSpec(
        num_scalar_prefetch=2,
        grid=(B // BB,),
        in_specs=[
            pl.BlockSpec((BB, 1, D), lambda b, *_: (b, 0, 0)),
            pl.BlockSpec((BB, 1, D), lambda b, *_: (b, 0, 0)),
            pl.BlockSpec((1, D), lambda b, *_: (0, 0)),
            pl.BlockSpec((1, D), lambda b, *_: (0, 0)),
            pl.BlockSpec((1, D), lambda b, *_: (0, 0)),
            pl.BlockSpec((1, D), lambda b, *_: (0, 0)),
            pl.BlockSpec((BB, S, D), lambda b, *_: (b, 0, 0)),
            pl.BlockSpec((BB, S, D), lambda b, *_: (b, 0, 0)),
        ],
        out_specs=[
            pl.BlockSpec((BB, 1, D), lambda b, *_: (b, 0, 0)),
            pl.BlockSpec((BB, 1, D), lambda b, *_: (b, 0, 0)),
            pl.BlockSpec((BB, S, D), lambda b, *_: (b, 0, 0)),
            pl.BlockSpec((BB, S, D), lambda b, *_: (b, 0, 0)),
        ],
    )
    read_r, read_i, next_r, next_i = pl.pallas_call(
        _stream_kernel,
        grid_spec=grid_spec,
        out_shape=(jax.ShapeDtypeStruct((B, 1, D), f32),
                   jax.ShapeDtypeStruct((B, 1, D), f32),
                   jax.ShapeDtypeStruct((B, S, D), f32),
                   jax.ShapeDtypeStruct((B, S, D), f32)),
        interpret=interpret,
    )(idx, euv, q_r, q_i, gamma_r.reshape(1, D), beta_r.reshape(1, D),
      gamma_i.reshape(1, D), beta_i.reshape(1, D), prev_mem_real, prev_mem_imag)

    return (read_r.reshape(B, D), read_i.reshape(B, D), next_r, next_i,
            ent.reshape(()))
